# baseline (speedup 1.0000x reference)
import functools

import jax
import jax.numpy as jnp
from jax import lax
from jax.experimental import pallas as pl
from jax.experimental.pallas import tpu as pltpu
from jax.experimental.pallas import tpu_sc as plsc

_B = 16384
_D = 64
_N_FEAT = 26
_FPT = 13
_OC = _FPT * _D


def _permute_rows():
    rows = []
    for i in range(_N_FEAT):
        rows.append((i // _FPT, i % 2, (i % _FPT) * _D, (i // 2) * _D))
    return tuple(rows)


_PERMUTES = _permute_rows()

_INFO = plsc.get_sparse_core_info()
_NC = _INFO.num_cores
_NS = _INFO.num_subcores
_NW = _NC * _NS
_RPW = _B // _NW

_mesh = plsc.VectorSubcoreMesh(core_axis_name="c", subcore_axis_name="s")


@functools.partial(
    pl.kernel,
    mesh=_mesh,
    out_type=(
        jax.ShapeDtypeStruct((_B, _OC), jnp.float32),
        jax.ShapeDtypeStruct((_B, _OC), jnp.float32),
    ),
    scratch_types=[pltpu.SemaphoreType.DMA],
    compiler_params=pltpu.CompilerParams(use_tc_tiling_on_sc=False),
)
def _permute_sc(v0, v1, o0, o1, sem):
    wid = lax.axis_index("s") * _NC + lax.axis_index("c")
    base = wid * _RPW
    ins = (v0, v1)
    outs = (o0, o1)
    copies = []
    for (ii, oi, istart, ostart) in _PERMUTES:
        c = pltpu.make_async_copy(
            ins[ii].at[pl.ds(base, _RPW), pl.ds(istart, _D)],
            outs[oi].at[pl.ds(base, _RPW), pl.ds(ostart, _D)],
            sem,
        )
        c.start()
        copies.append(c)
    for c in copies:
        c.wait()


@jax.jit
def kernel(values_0, values_1):
    return _permute_sc(values_0, values_1)


# speedup vs baseline: 7.7577x; 7.7577x over previous
import functools

import jax
import jax.numpy as jnp
from jax import lax
from jax.experimental import pallas as pl
from jax.experimental.pallas import tpu as pltpu
from jax.experimental.pallas import tpu_sc as plsc

_B = 16384
_D = 64
_N_FEAT = 26
_FPT = 13
_OC = _FPT * _D
_LANES = 16
_VPC = _D // _LANES


def _permute_rows():
    rows = []
    for i in range(_N_FEAT):
        rows.append((i // _FPT, i % 2, (i % _FPT) * _D, (i // 2) * _D))
    return tuple(rows)


_PERMUTES = _permute_rows()

_INFO = plsc.get_sparse_core_info()
_NC = _INFO.num_cores
_NS = _INFO.num_subcores
_NW = _NC * _NS
_RPW = _B // _NW

_R = 16
_NBLK = _RPW // _R

_mesh = plsc.VectorSubcoreMesh(core_axis_name="c", subcore_axis_name="s")


@functools.partial(
    pl.kernel,
    mesh=_mesh,
    out_type=(
        jax.ShapeDtypeStruct((_B, _OC), jnp.float32),
        jax.ShapeDtypeStruct((_B, _OC), jnp.float32),
    ),
    scratch_types=(
        [pltpu.VMEM((_R, _OC), jnp.float32) for _ in range(8)]
        + [pltpu.SemaphoreType.DMA for _ in range(4)]
    ),
    compiler_params=pltpu.CompilerParams(use_tc_tiling_on_sc=False),
)
def _permute_sc(v0, v1, o0, o1,
                in0a, in0b, in1a, in1b, out0a, out0b, out1a, out1b,
                sem_in_a, sem_in_b, sem_out_a, sem_out_b):
    wid = lax.axis_index("s") * _NC + lax.axis_index("c")
    base = wid * _RPW

    inb = ((in0a, in1a), (in0b, in1b))
    outb = ((out0a, out1a), (out0b, out1b))
    sem_in = (sem_in_a, sem_in_b)
    sem_out = (sem_out_a, sem_out_b)
    ins = (v0, v1)
    outs = (o0, o1)

    def in_copies(t, s):
        rows = pl.ds(base + t * _R, _R)
        return [
            pltpu.make_async_copy(ins[i].at[rows, :], inb[s][i], sem_in[s])
            for i in range(2)
        ]

    def out_copies(t, s):
        rows = pl.ds(base + t * _R, _R)
        return [
            pltpu.make_async_copy(outb[s][i], outs[i].at[rows, :], sem_out[s])
            for i in range(2)
        ]

    def start(copies):
        for c in copies:
            c.start()

    def wait(copies):
        for c in copies:
            c.wait()

    def regroup(s):
        src0, src1 = inb[s]
        dsts = outb[s]
        srcs = (src0, src1)

        @pl.loop(0, _R)
        def _(r):
            for (ii, oi, istart, ostart) in _PERMUTES:
                for k in range(_VPC):
                    dsts[oi][r, pl.ds(ostart + k * _LANES, _LANES)] = (
                        srcs[ii][r, pl.ds(istart + k * _LANES, _LANES)]
                    )

    start(in_copies(0, 0))
    start(in_copies(1, 1))

    for s, t in ((0, 0), (1, 1)):
        wait(in_copies(t, s))
        regroup(s)
        start(out_copies(t, s))
        start(in_copies(t + 2, s))

    @pl.loop(1, _NBLK // 2 - 1)
    def _(g):
        for s in (0, 1):
            t = 2 * g + s
            wait(in_copies(t, s))
            wait(out_copies(t - 2, s))
            regroup(s)
            start(out_copies(t, s))
            start(in_copies(t + 2, s))

    for s in (0, 1):
        t = _NBLK - 2 + s
        wait(in_copies(t, s))
        wait(out_copies(t - 2, s))
        regroup(s)
        start(out_copies(t, s))

    for s in (0, 1):
        wait(out_copies(_NBLK - 2 + s, s))


@jax.jit
def kernel(values_0, values_1):
    return _permute_sc(values_0, values_1)


# speedup vs baseline: 11.4387x; 1.4745x over previous
import functools

import jax
import jax.numpy as jnp
from jax import lax
from jax.experimental import pallas as pl
from jax.experimental.pallas import tpu as pltpu
from jax.experimental.pallas import tpu_sc as plsc

_B = 16384
_D = 64
_N_FEAT = 26
_FPT = 13
_OC = _FPT * _D
_LANES = 16
_VPC = _D // _LANES


def _permute_rows():
    rows = []
    for i in range(_N_FEAT):
        rows.append((i // _FPT, i % 2, (i % _FPT) * _D, (i // 2) * _D))
    return tuple(rows)


_PERMUTES = _permute_rows()

_INFO = plsc.get_sparse_core_info()
_NC = _INFO.num_cores
_NS = _INFO.num_subcores
_NW = _NC * _NS
_RPW = _B // _NW

_R = 16
_NBLK = _RPW // _R

_mesh = plsc.VectorSubcoreMesh(core_axis_name="c", subcore_axis_name="s")


@functools.partial(
    pl.kernel,
    mesh=_mesh,
    out_type=(
        jax.ShapeDtypeStruct((_B, _OC), jnp.float32),
        jax.ShapeDtypeStruct((_B, _OC), jnp.float32),
    ),
    scratch_types=(
        [pltpu.VMEM((_R, _OC), jnp.float32) for _ in range(8)]
        + [pltpu.SemaphoreType.DMA for _ in range(4)]
    ),
)
def _permute_sc(v0, v1, o0, o1,
                in0a, in0b, in1a, in1b, out0a, out0b, out1a, out1b,
                sem_in_a, sem_in_b, sem_out_a, sem_out_b):
    wid = lax.axis_index("s") * _NC + lax.axis_index("c")
    base = wid * _RPW

    inb = ((in0a, in1a), (in0b, in1b))
    outb = ((out0a, out1a), (out0b, out1b))
    sem_in = (sem_in_a, sem_in_b)
    sem_out = (sem_out_a, sem_out_b)
    ins = (v0, v1)
    outs = (o0, o1)

    def in_copies(t, s):
        rows = pl.ds(base + t * _R, _R)
        return [
            pltpu.make_async_copy(ins[i].at[rows, :], inb[s][i], sem_in[s])
            for i in range(2)
        ]

    def out_copies(t, s):
        rows = pl.ds(base + t * _R, _R)
        return [
            pltpu.make_async_copy(outb[s][i], outs[i].at[rows, :], sem_out[s])
            for i in range(2)
        ]

    def start(copies):
        for c in copies:
            c.start()

    def wait(copies):
        for c in copies:
            c.wait()

    def regroup(s):
        src0, src1 = inb[s]
        dsts = outb[s]
        srcs = (src0, src1)

        @pl.loop(0, _R)
        def _(r):
            for (ii, oi, istart, ostart) in _PERMUTES:
                for k in range(_VPC):
                    dsts[oi][r, pl.ds(ostart + k * _LANES, _LANES)] = (
                        srcs[ii][r, pl.ds(istart + k * _LANES, _LANES)]
                    )

    start(in_copies(0, 0))
    start(in_copies(1, 1))

    for s, t in ((0, 0), (1, 1)):
        wait(in_copies(t, s))
        regroup(s)
        start(out_copies(t, s))
        start(in_copies(t + 2, s))

    @pl.loop(1, _NBLK // 2 - 1)
    def _(g):
        for s in (0, 1):
            t = 2 * g + s
            wait(in_copies(t, s))
            wait(out_copies(t - 2, s))
            regroup(s)
            start(out_copies(t, s))
            start(in_copies(t + 2, s))

    for s in (0, 1):
        t = _NBLK - 2 + s
        wait(in_copies(t, s))
        wait(out_copies(t - 2, s))
        regroup(s)
        start(out_copies(t, s))

    for s in (0, 1):
        wait(out_copies(_NBLK - 2 + s, s))


@jax.jit
def kernel(values_0, values_1):
    return _permute_sc(values_0, values_1)


# speedup vs baseline: 37.9854x; 3.3208x over previous
import functools

import jax
import jax.numpy as jnp
from jax import lax
from jax.experimental import pallas as pl
from jax.experimental.pallas import tpu as pltpu
from jax.experimental.pallas import tpu_sc as plsc

_B = 16384
_D = 64
_N_FEAT = 26
_FPT = 13
_OC = _FPT * _D

_PERMUTES = tuple(
    (i // _FPT, i % 2, (i % _FPT) * _D, (i // 2) * _D) for i in range(_N_FEAT)
)

_INFO = plsc.get_sparse_core_info()
_NC = _INFO.num_cores
_NS = _INFO.num_subcores
_NW = _NC * _NS
_CW = _B // _NW

_NBUF = 3

_mesh = plsc.VectorSubcoreMesh(core_axis_name="c", subcore_axis_name="s")


@functools.partial(
    pl.kernel,
    mesh=_mesh,
    compiler_params=pltpu.CompilerParams(use_tc_tiling_on_sc=True),
    out_type=(
        jax.ShapeDtypeStruct((_OC, _B), jnp.float32),
        jax.ShapeDtypeStruct((_OC, _B), jnp.float32),
    ),
    scratch_types=(
        [pltpu.VMEM((_D, _CW), jnp.float32) for _ in range(_NBUF)]
        + [pltpu.SemaphoreType.DMA for _ in range(2 * _NBUF)]
    ),
)
def _permute_sc(v0, v1, o0, o1, buf0, buf1, buf2, sg0, sg1, sg2, ss0, ss1, ss2):
    bufs = (buf0, buf1, buf2)
    sem_g = (sg0, sg1, sg2)
    sem_s = (ss0, ss1, ss2)
    ins = (v0, v1)
    outs = (o0, o1)

    wid = lax.axis_index("s") * _NC + lax.axis_index("c")
    cols = pl.ds(wid * _CW, _CW)

    def g_copy(f, s):
        ii, _, istart, _ = _PERMUTES[f]
        return pltpu.make_async_copy(
            ins[ii].at[pl.ds(istart, _D), cols], bufs[s], sem_g[s]
        )

    def s_copy(f, s):
        _, oi, _, ostart = _PERMUTES[f]
        return pltpu.make_async_copy(
            bufs[s], outs[oi].at[pl.ds(ostart, _D), cols], sem_s[s]
        )

    g_copy(0, 0).start()
    g_copy(1, 1).start()
    for f in range(_N_FEAT):
        s = f % _NBUF
        g_copy(f, s).wait()
        s_copy(f, s).start()
        if f + 2 < _N_FEAT:
            nxt = (f + 2) % _NBUF
            if f - 1 >= 0:
                s_copy(f - 1, nxt).wait()
            g_copy(f + 2, nxt).start()
    s_copy(_N_FEAT - 2, (_N_FEAT - 2) % _NBUF).wait()
    s_copy(_N_FEAT - 1, (_N_FEAT - 1) % _NBUF).wait()


@jax.jit
def kernel(values_0, values_1):
    o0t, o1t = _permute_sc(values_0.T, values_1.T)
    return o0t.T, o1t.T
